# baseline (speedup 1.0000x reference)
import functools

import jax
import jax.numpy as jnp
from jax import lax
from jax.experimental import pallas as pl
from jax.experimental.pallas import tpu as pltpu

N = 10000
E = 320000
D_IN = 128
HGC = 16
EDGENET_DIM = 16
PAE_IN = EDGENET_DIM // 2
PAE_HID = 128
BN_EPS = 1e-5

_PAE_BLK = 512


def _pae_body(ein_ref, w1_ref, b1_ref, w2_ref, b2_ref, g_ref, b_ref, out_ref):
    ein = ein_ref[...]
    w1 = w1_ref[...]
    w2 = w2_ref[...]
    b1 = b1_ref[...]
    b2 = b2_ref[...]
    scale = g_ref[...] * (1.0 / jnp.sqrt(1.0 + BN_EPS))
    shift = b_ref[...]

    def tower(x):
        h = jnp.maximum(jnp.dot(x, w1, preferred_element_type=jnp.float32) + b1, 0.0)
        h = h * scale + shift
        return jnp.dot(h, w2, preferred_element_type=jnp.float32) + b2

    y1 = tower(ein[:, :PAE_IN])
    y2 = tower(ein[:, PAE_IN:])
    num = jnp.sum(y1 * y2, axis=1)
    n1 = jnp.maximum(jnp.sqrt(jnp.sum(y1 * y1, axis=1)), 1e-8)
    n2 = jnp.maximum(jnp.sqrt(jnp.sum(y2 * y2, axis=1)), 1e-8)
    out_ref[...] = (num / (n1 * n2) + 1.0) * 0.5


def _pae_edge_weight(ein, w1, b1, w2, b2, g, b):
    grid = (E // _PAE_BLK,)
    return pl.pallas_call(
        _pae_body,
        grid=grid,
        in_specs=[
            pl.BlockSpec((_PAE_BLK, EDGENET_DIM), lambda i: (i, 0)),
            pl.BlockSpec((PAE_IN, PAE_HID), lambda i: (0, 0)),
            pl.BlockSpec((PAE_HID,), lambda i: (0,)),
            pl.BlockSpec((PAE_HID, PAE_HID), lambda i: (0, 0)),
            pl.BlockSpec((PAE_HID,), lambda i: (0,)),
            pl.BlockSpec((PAE_HID,), lambda i: (0,)),
            pl.BlockSpec((PAE_HID,), lambda i: (0,)),
        ],
        out_specs=pl.BlockSpec((_PAE_BLK,), lambda i: (i,)),
        out_shape=jax.ShapeDtypeStruct((E,), jnp.float32),
    )(ein, w1, b1, w2, b2, g, b)


def _prop(y, src, dst, norm):
    return jnp.zeros_like(y).at[dst].add(norm[:, None] * y[src])


def kernel(features, edge_index, edgenet_input, pae_w1, pae_b1, pae_w2, pae_b2,
           pae_bn_g, pae_bn_b, cheb0_w, cheb1_w, cheb2_w, cheb3_w,
           cls_w1, cls_b1, cls_bn_g, cls_bn_b, cls_w2, cls_b2):
    src = edge_index[0]
    dst = edge_index[1]

    edge_weight = _pae_edge_weight(edgenet_input, pae_w1, pae_b1, pae_w2,
                                   pae_b2, pae_bn_g, pae_bn_b)

    w = jnp.where(src == dst, 0.0, edge_weight)
    deg = jnp.zeros((N,), jnp.float32).at[src].add(w)
    dis = jnp.where(deg > 0, lax.rsqrt(jnp.maximum(deg, 1e-12)), 0.0)
    norm = -dis[src] * w * dis[dst]

    mm = functools.partial(jnp.matmul, precision=lax.Precision.HIGHEST)
    h = features
    h0_parts = []
    for W in (cheb0_w, cheb1_w, cheb2_w, cheb3_w):
        z0 = mm(h, W[0] - W[2])
        z1 = mm(h, W[1])
        z2 = mm(h, W[2])
        r = _prop(z2, src, dst, norm)
        t = _prop(z1 + 2.0 * r, src, dst, norm)
        h = jnp.maximum(z0 + t, 0.0)
        h0_parts.append(h)
    h0 = jnp.concatenate(h0_parts, axis=1)

    z = jnp.maximum(mm(h0, cls_w1) + cls_b1, 0.0)
    z = z * cls_bn_g * (1.0 / jnp.sqrt(1.0 + BN_EPS)) + cls_bn_b
    logit = mm(z, cls_w2) + cls_b2
    return (logit, edge_weight)


# speedup vs baseline: 10.2452x; 10.2452x over previous
import functools

import jax
import jax.numpy as jnp
from jax import lax
from jax.experimental import pallas as pl
from jax.experimental.pallas import tpu as pltpu
from jax.experimental.pallas import tpu_sc as plsc

NC = 2
NS = 16
NW = NC * NS
L = 16

N = 10000
E = 320000
D_IN = 128
HGC = 16
EDGENET_DIM = 16
PAE_IN = EDGENET_DIM // 2
PAE_HID = 128
BN_EPS = 1e-5

_PAE_BLK = 512


def _pae_body(ein_ref, w1_ref, b1_ref, w2_ref, b2_ref, g_ref, b_ref, out_ref):
    ein = ein_ref[...]
    w1 = w1_ref[...]
    w2 = w2_ref[...]
    b1 = b1_ref[...]
    b2 = b2_ref[...]
    scale = g_ref[...] * (1.0 / jnp.sqrt(1.0 + BN_EPS))
    shift = b_ref[...]

    def tower(x):
        h = jnp.maximum(jnp.dot(x, w1, preferred_element_type=jnp.float32) + b1, 0.0)
        h = h * scale + shift
        return jnp.dot(h, w2, preferred_element_type=jnp.float32) + b2

    y1 = tower(ein[:, :PAE_IN])
    y2 = tower(ein[:, PAE_IN:])
    num = jnp.sum(y1 * y2, axis=1)
    n1 = jnp.maximum(jnp.sqrt(jnp.sum(y1 * y1, axis=1)), 1e-8)
    n2 = jnp.maximum(jnp.sqrt(jnp.sum(y2 * y2, axis=1)), 1e-8)
    out_ref[...] = (num / (n1 * n2) + 1.0) * 0.5


def _pae_edge_weight(ein, w1, b1, w2, b2, g, b):
    grid = (E // _PAE_BLK,)
    return pl.pallas_call(
        _pae_body,
        grid=grid,
        in_specs=[
            pl.BlockSpec((_PAE_BLK, EDGENET_DIM), lambda i: (i, 0)),
            pl.BlockSpec((PAE_IN, PAE_HID), lambda i: (0, 0)),
            pl.BlockSpec((PAE_HID,), lambda i: (0,)),
            pl.BlockSpec((PAE_HID, PAE_HID), lambda i: (0, 0)),
            pl.BlockSpec((PAE_HID,), lambda i: (0,)),
            pl.BlockSpec((PAE_HID,), lambda i: (0,)),
            pl.BlockSpec((PAE_HID,), lambda i: (0,)),
        ],
        out_specs=pl.BlockSpec((_PAE_BLK,), lambda i: (i,)),
        out_shape=jax.ShapeDtypeStruct((E,), jnp.float32),
    )(ein, w1, b1, w2, b2, g, b)


_ECH = E // NW
_NVR = _ECH // L
_NSL = N // NS

_MESH = plsc.VectorSubcoreMesh(core_axis_name="c", subcore_axis_name="s",
                               num_cores=NC, num_subcores=NS)


def _wid():
    return lax.axis_index("s") * NC + lax.axis_index("c")


def _deg_body(src_hbm, dst_hbm, ew_hbm, out_hbm, src_v, dst_v, ew_v, deg_v):
    wid = _wid()
    zero = jnp.zeros((L,), jnp.float32)

    def zbody(k, _):
        deg_v[pl.ds(k * L, L)] = zero
        return 0

    lax.fori_loop(0, N // L, zbody, 0)
    base = wid * _ECH
    pltpu.sync_copy(src_hbm.at[pl.ds(base, _ECH)], src_v)
    pltpu.sync_copy(dst_hbm.at[pl.ds(base, _ECH)], dst_v)
    pltpu.sync_copy(ew_hbm.at[pl.ds(base, _ECH)], ew_v)

    def body(k, _):
        sl = pl.ds(k * L, L)
        s = src_v[sl]
        w = jnp.where(s == dst_v[sl], 0.0, ew_v[sl])
        plsc.addupdate_scatter(deg_v, [s], w)
        return 0

    lax.fori_loop(0, _NVR, body, 0)
    pltpu.sync_copy(deg_v, out_hbm.at[wid])


@functools.partial(
    pl.kernel,
    out_type=jax.ShapeDtypeStruct((NW, N), jnp.float32),
    mesh=_MESH,
    compiler_params=pltpu.CompilerParams(needs_layout_passes=False, use_tc_tiling_on_sc=False),
    scratch_types=[
        pltpu.VMEM((_ECH,), jnp.int32),
        pltpu.VMEM((_ECH,), jnp.int32),
        pltpu.VMEM((_ECH,), jnp.float32),
        pltpu.VMEM((N,), jnp.float32),
    ],
)
def _sc_degree(src_hbm, dst_hbm, ew_hbm, out_hbm, src_v, dst_v, ew_v, deg_v):
    _deg_body(src_hbm, dst_hbm, ew_hbm, out_hbm, src_v, dst_v, ew_v, deg_v)


@functools.partial(
    pl.kernel,
    out_type=jax.ShapeDtypeStruct((E,), jnp.float32),
    mesh=_MESH,
    compiler_params=pltpu.CompilerParams(needs_layout_passes=False, use_tc_tiling_on_sc=False),
    scratch_types=[
        pltpu.VMEM((_ECH,), jnp.int32),
        pltpu.VMEM((_ECH,), jnp.int32),
        pltpu.VMEM((_ECH,), jnp.float32),
        pltpu.VMEM((N,), jnp.float32),
        pltpu.VMEM((_ECH,), jnp.float32),
    ],
)
def _sc_norm(src_hbm, dst_hbm, ew_hbm, dis_hbm, out_hbm,
             src_v, dst_v, ew_v, dis_v, nrm_v):
    wid = _wid()
    base = wid * _ECH
    pltpu.sync_copy(src_hbm.at[pl.ds(base, _ECH)], src_v)
    pltpu.sync_copy(dst_hbm.at[pl.ds(base, _ECH)], dst_v)
    pltpu.sync_copy(ew_hbm.at[pl.ds(base, _ECH)], ew_v)
    pltpu.sync_copy(dis_hbm, dis_v)

    def body(k, _):
        sl = pl.ds(k * L, L)
        s = src_v[sl]
        d = dst_v[sl]
        w = jnp.where(s == d, 0.0, ew_v[sl])
        a = plsc.load_gather(dis_v, [s])
        b = plsc.load_gather(dis_v, [d])
        nrm_v[sl] = -(a * w * b)
        return 0

    lax.fori_loop(0, _NVR, body, 0)
    pltpu.sync_copy(nrm_v, out_hbm.at[pl.ds(base, _ECH)])


_PCH = 2000
_PNCH = _ECH // _PCH


@functools.partial(
    pl.kernel,
    out_type=jax.ShapeDtypeStruct((NC, N, HGC), jnp.float32),
    mesh=_MESH,
    compiler_params=pltpu.CompilerParams(needs_layout_passes=False, use_tc_tiling_on_sc=False),
    scratch_types=[
        pltpu.VMEM((_PCH,), jnp.int32),
        pltpu.VMEM((_PCH,), jnp.int32),
        pltpu.VMEM((_PCH,), jnp.float32),
        pltpu.VMEM((_PCH, HGC), jnp.float32),
        pltpu.VMEM_SHARED((N, HGC), jnp.float32),
        pltpu.VMEM_SHARED((N, HGC), jnp.float32),
        pltpu.SemaphoreType.DMA,
    ],
)
def _sc_prop(y_hbm, src_hbm, dst_hbm, nrm_hbm, zeros_hbm, out_hbm,
             src_v, dst_v, nrm_v, rows_v, acc_sh, y_sh, sem):
    cid = lax.axis_index("c")
    sid = lax.axis_index("s")
    wid = sid * NC + cid
    zp = 624
    rem = N - NS * zp
    off = pl.multiple_of(sid * zp, 8)
    rsl = pl.ds(off, zp)
    tail = pl.ds(NS * zp, rem)
    pltpu.sync_copy(zeros_hbm.at[rsl], acc_sh.at[rsl])
    pltpu.sync_copy(y_hbm.at[rsl], y_sh.at[rsl])

    @pl.when(sid == 0)
    def _zero_tail():
        pltpu.sync_copy(zeros_hbm.at[tail], acc_sh.at[tail])
        pltpu.sync_copy(y_hbm.at[tail], y_sh.at[tail])

    plsc.subcore_barrier()

    for j in range(_PNCH):
        base = wid * _ECH + j * _PCH
        pltpu.sync_copy(src_hbm.at[pl.ds(base, _PCH)], src_v)
        pltpu.sync_copy(dst_hbm.at[pl.ds(base, _PCH)], dst_v)
        pltpu.sync_copy(nrm_hbm.at[pl.ds(base, _PCH)], nrm_v)
        pltpu.async_copy(y_sh.at[src_v], rows_v, sem).wait()

        cols = lax.iota(jnp.int32, L)

        def body(g, _):
            for i in range(L):
                e = g * L + i
                esplat = jnp.full((L,), e, dtype=jnp.int32)
                nsplat = plsc.load_gather(nrm_v, [esplat])
                v = plsc.load_gather(rows_v, [esplat, cols])
                plsc.store_scatter(rows_v, [esplat, cols], v * nsplat)
            return 0

        lax.fori_loop(0, _PCH // L, body, 0)
        pltpu.sync_copy(rows_v, acc_sh.at[dst_v], add=True)

    plsc.subcore_barrier()
    pltpu.sync_copy(acc_sh.at[rsl], out_hbm.at[cid, rsl])

    @pl.when(sid == 0)
    def _write_tail():
        pltpu.sync_copy(acc_sh.at[tail], out_hbm.at[cid, tail])


def _prop(y, src, dst, norm, zeros_n16):
    parts = _sc_prop(y, src, dst, norm, zeros_n16)
    return parts[0] + parts[1]


def kernel(features, edge_index, edgenet_input, pae_w1, pae_b1, pae_w2, pae_b2,
           pae_bn_g, pae_bn_b, cheb0_w, cheb1_w, cheb2_w, cheb3_w,
           cls_w1, cls_b1, cls_bn_g, cls_bn_b, cls_w2, cls_b2):
    src = edge_index[0]
    dst = edge_index[1]

    edge_weight = _pae_edge_weight(edgenet_input, pae_w1, pae_b1, pae_w2,
                                   pae_b2, pae_bn_g, pae_bn_b)

    deg = jnp.sum(_sc_degree(src, dst, edge_weight), axis=0)
    dis = jnp.where(deg > 0, lax.rsqrt(jnp.maximum(deg, 1e-12)), 0.0)
    norm = _sc_norm(src, dst, edge_weight, dis)
    zeros_n16 = jnp.zeros((N, HGC), jnp.float32)

    mm = functools.partial(jnp.matmul, precision=lax.Precision.HIGHEST)
    h = features
    h0_parts = []
    for W in (cheb0_w, cheb1_w, cheb2_w, cheb3_w):
        z0 = mm(h, W[0] - W[2])
        z1 = mm(h, W[1])
        z2 = mm(h, W[2])
        r = _prop(z2, src, dst, norm, zeros_n16)
        t = _prop(z1 + 2.0 * r, src, dst, norm, zeros_n16)
        h = jnp.maximum(z0 + t, 0.0)
        h0_parts.append(h)
    h0 = jnp.concatenate(h0_parts, axis=1)

    z = jnp.maximum(mm(h0, cls_w1) + cls_b1, 0.0)
    z = z * cls_bn_g * (1.0 / jnp.sqrt(1.0 + BN_EPS)) + cls_bn_b
    logit = mm(z, cls_w2) + cls_b2
    return (logit, edge_weight)


# speedup vs baseline: 13.3531x; 1.3034x over previous
import functools

import jax
import jax.numpy as jnp
from jax import lax
from jax.experimental import pallas as pl
from jax.experimental.pallas import tpu as pltpu
from jax.experimental.pallas import tpu_sc as plsc

NC = 2
NS = 16
NW = NC * NS
L = 16

N = 10000
E = 320000
D_IN = 128
HGC = 16
EDGENET_DIM = 16
PAE_IN = EDGENET_DIM // 2
PAE_HID = 128
BN_EPS = 1e-5

_PAE_BLK = 512


def _pae_body(ein_ref, w1_ref, b1_ref, w2_ref, b2_ref, g_ref, b_ref, out_ref):
    ein = ein_ref[...]
    w1 = w1_ref[...]
    w2 = w2_ref[...]
    b1 = b1_ref[...]
    b2 = b2_ref[...]
    scale = g_ref[...] * (1.0 / jnp.sqrt(1.0 + BN_EPS))
    shift = b_ref[...]

    def tower(x):
        h = jnp.maximum(jnp.dot(x, w1, preferred_element_type=jnp.float32) + b1, 0.0)
        h = h * scale + shift
        return jnp.dot(h, w2, preferred_element_type=jnp.float32) + b2

    y1 = tower(ein[:, :PAE_IN])
    y2 = tower(ein[:, PAE_IN:])
    num = jnp.sum(y1 * y2, axis=1)
    n1 = jnp.maximum(jnp.sqrt(jnp.sum(y1 * y1, axis=1)), 1e-8)
    n2 = jnp.maximum(jnp.sqrt(jnp.sum(y2 * y2, axis=1)), 1e-8)
    out_ref[...] = (num / (n1 * n2) + 1.0) * 0.5


def _pae_edge_weight(ein, w1, b1, w2, b2, g, b):
    grid = (E // _PAE_BLK,)
    return pl.pallas_call(
        _pae_body,
        grid=grid,
        in_specs=[
            pl.BlockSpec((_PAE_BLK, EDGENET_DIM), lambda i: (i, 0)),
            pl.BlockSpec((PAE_IN, PAE_HID), lambda i: (0, 0)),
            pl.BlockSpec((PAE_HID,), lambda i: (0,)),
            pl.BlockSpec((PAE_HID, PAE_HID), lambda i: (0, 0)),
            pl.BlockSpec((PAE_HID,), lambda i: (0,)),
            pl.BlockSpec((PAE_HID,), lambda i: (0,)),
            pl.BlockSpec((PAE_HID,), lambda i: (0,)),
        ],
        out_specs=pl.BlockSpec((_PAE_BLK,), lambda i: (i,)),
        out_shape=jax.ShapeDtypeStruct((E,), jnp.float32),
    )(ein, w1, b1, w2, b2, g, b)


_ECH = E // NW
_NVR = _ECH // L
_NSL = N // NS

_MESH = plsc.VectorSubcoreMesh(core_axis_name="c", subcore_axis_name="s",
                               num_cores=NC, num_subcores=NS)


def _wid():
    return lax.axis_index("s") * NC + lax.axis_index("c")


def _deg_body(src_hbm, dst_hbm, ew_hbm, out_hbm, src_v, dst_v, ew_v, deg_v):
    wid = _wid()
    zero = jnp.zeros((L,), jnp.float32)

    def zbody(k, _):
        deg_v[pl.ds(k * L, L)] = zero
        return 0

    lax.fori_loop(0, N // L, zbody, 0)
    base = wid * _ECH
    pltpu.sync_copy(src_hbm.at[pl.ds(base, _ECH)], src_v)
    pltpu.sync_copy(dst_hbm.at[pl.ds(base, _ECH)], dst_v)
    pltpu.sync_copy(ew_hbm.at[pl.ds(base, _ECH)], ew_v)

    def body(k, _):
        sl = pl.ds(k * L, L)
        s = src_v[sl]
        w = jnp.where(s == dst_v[sl], 0.0, ew_v[sl])
        plsc.addupdate_scatter(deg_v, [s], w)
        return 0

    lax.fori_loop(0, _NVR, body, 0)
    pltpu.sync_copy(deg_v, out_hbm.at[wid])


@functools.partial(
    pl.kernel,
    out_type=jax.ShapeDtypeStruct((NW, N), jnp.float32),
    mesh=_MESH,
    compiler_params=pltpu.CompilerParams(needs_layout_passes=False, use_tc_tiling_on_sc=False),
    scratch_types=[
        pltpu.VMEM((_ECH,), jnp.int32),
        pltpu.VMEM((_ECH,), jnp.int32),
        pltpu.VMEM((_ECH,), jnp.float32),
        pltpu.VMEM((N,), jnp.float32),
    ],
)
def _sc_degree(src_hbm, dst_hbm, ew_hbm, out_hbm, src_v, dst_v, ew_v, deg_v):
    _deg_body(src_hbm, dst_hbm, ew_hbm, out_hbm, src_v, dst_v, ew_v, deg_v)


@functools.partial(
    pl.kernel,
    out_type=jax.ShapeDtypeStruct((E,), jnp.float32),
    mesh=_MESH,
    compiler_params=pltpu.CompilerParams(needs_layout_passes=False, use_tc_tiling_on_sc=False),
    scratch_types=[
        pltpu.VMEM((_ECH,), jnp.int32),
        pltpu.VMEM((_ECH,), jnp.int32),
        pltpu.VMEM((_ECH,), jnp.float32),
        pltpu.VMEM((N,), jnp.float32),
        pltpu.VMEM((_ECH,), jnp.float32),
    ],
)
def _sc_norm(src_hbm, dst_hbm, ew_hbm, dis_hbm, out_hbm,
             src_v, dst_v, ew_v, dis_v, nrm_v):
    wid = _wid()
    base = wid * _ECH
    pltpu.sync_copy(src_hbm.at[pl.ds(base, _ECH)], src_v)
    pltpu.sync_copy(dst_hbm.at[pl.ds(base, _ECH)], dst_v)
    pltpu.sync_copy(ew_hbm.at[pl.ds(base, _ECH)], ew_v)
    pltpu.sync_copy(dis_hbm, dis_v)

    def body(k, _):
        sl = pl.ds(k * L, L)
        s = src_v[sl]
        d = dst_v[sl]
        w = jnp.where(s == d, 0.0, ew_v[sl])
        a = plsc.load_gather(dis_v, [s])
        b = plsc.load_gather(dis_v, [d])
        nrm_v[sl] = -(a * w * b)
        return 0

    lax.fori_loop(0, _NVR, body, 0)
    pltpu.sync_copy(nrm_v, out_hbm.at[pl.ds(base, _ECH)])


_PCH = 2000
_PNCH = _ECH // _PCH


@functools.partial(
    pl.kernel,
    out_type=jax.ShapeDtypeStruct((NC, N, HGC), jnp.float32),
    mesh=_MESH,
    compiler_params=pltpu.CompilerParams(needs_layout_passes=False, use_tc_tiling_on_sc=False),
    scratch_types=[
        pltpu.VMEM((_PCH,), jnp.int32),
        pltpu.VMEM((_PCH,), jnp.int32),
        pltpu.VMEM((_PCH,), jnp.float32),
        pltpu.VMEM((_PCH, HGC), jnp.float32),
        pltpu.VMEM_SHARED((N, HGC), jnp.float32),
        pltpu.VMEM_SHARED((N, HGC), jnp.float32),
        pltpu.SemaphoreType.DMA,
    ],
)
def _sc_prop(y_hbm, src_hbm, dst_hbm, nrm_hbm, zeros_hbm, out_hbm,
             src_v, dst_v, nrm_v, rows_v, acc_sh, y_sh, sem):
    cid = lax.axis_index("c")
    sid = lax.axis_index("s")
    wid = sid * NC + cid
    zp = 624
    rem = N - NS * zp
    off = pl.multiple_of(sid * zp, 8)
    rsl = pl.ds(off, zp)
    tail = pl.ds(NS * zp, rem)
    pltpu.sync_copy(zeros_hbm.at[rsl], acc_sh.at[rsl])
    pltpu.sync_copy(y_hbm.at[rsl], y_sh.at[rsl])

    @pl.when(sid == 0)
    def _zero_tail():
        pltpu.sync_copy(zeros_hbm.at[tail], acc_sh.at[tail])
        pltpu.sync_copy(y_hbm.at[tail], y_sh.at[tail])

    plsc.subcore_barrier()

    for j in range(_PNCH):
        base = wid * _ECH + j * _PCH
        pltpu.sync_copy(src_hbm.at[pl.ds(base, _PCH)], src_v)
        pltpu.sync_copy(dst_hbm.at[pl.ds(base, _PCH)], dst_v)
        pltpu.sync_copy(nrm_hbm.at[pl.ds(base, _PCH)], nrm_v)
        pltpu.async_copy(y_sh.at[src_v], rows_v, sem).wait()

        def body(g, _):
            nv = nrm_v[pl.ds(g * L, L)]
            for i in range(L):
                e = g * L + i
                rows_v[e] = rows_v[e] * nv[i]
            return 0

        lax.fori_loop(0, _PCH // L, body, 0)
        pltpu.sync_copy(rows_v, acc_sh.at[dst_v], add=True)

    plsc.subcore_barrier()
    pltpu.sync_copy(acc_sh.at[rsl], out_hbm.at[cid, rsl])

    @pl.when(sid == 0)
    def _write_tail():
        pltpu.sync_copy(acc_sh.at[tail], out_hbm.at[cid, tail])


def _prop(y, src, dst, norm, zeros_n16):
    parts = _sc_prop(y, src, dst, norm, zeros_n16)
    return parts[0] + parts[1]


def kernel(features, edge_index, edgenet_input, pae_w1, pae_b1, pae_w2, pae_b2,
           pae_bn_g, pae_bn_b, cheb0_w, cheb1_w, cheb2_w, cheb3_w,
           cls_w1, cls_b1, cls_bn_g, cls_bn_b, cls_w2, cls_b2):
    src = edge_index[0]
    dst = edge_index[1]

    edge_weight = _pae_edge_weight(edgenet_input, pae_w1, pae_b1, pae_w2,
                                   pae_b2, pae_bn_g, pae_bn_b)

    deg = jnp.sum(_sc_degree(src, dst, edge_weight), axis=0)
    dis = jnp.where(deg > 0, lax.rsqrt(jnp.maximum(deg, 1e-12)), 0.0)
    norm = _sc_norm(src, dst, edge_weight, dis)
    zeros_n16 = jnp.zeros((N, HGC), jnp.float32)

    mm = functools.partial(jnp.matmul, precision=lax.Precision.HIGHEST)
    h = features
    h0_parts = []
    for W in (cheb0_w, cheb1_w, cheb2_w, cheb3_w):
        z0 = mm(h, W[0] - W[2])
        z1 = mm(h, W[1])
        z2 = mm(h, W[2])
        r = _prop(z2, src, dst, norm, zeros_n16)
        t = _prop(z1 + 2.0 * r, src, dst, norm, zeros_n16)
        h = jnp.maximum(z0 + t, 0.0)
        h0_parts.append(h)
    h0 = jnp.concatenate(h0_parts, axis=1)

    z = jnp.maximum(mm(h0, cls_w1) + cls_b1, 0.0)
    z = z * cls_bn_g * (1.0 / jnp.sqrt(1.0 + BN_EPS)) + cls_bn_b
    logit = mm(z, cls_w2) + cls_b2
    return (logit, edge_weight)
